# baseline (speedup 1.0000x reference)
import functools

import jax
import jax.numpy as jnp
from jax import lax
from jax.experimental import pallas as pl
from jax.experimental.pallas import tpu as pltpu
from jax.experimental.pallas import tpu_sc as plsc

N = 2048
E = 131072
D = 128
H = 128

NC = 2
NS = 16
CW = 128
RH = 512
NCC = N // CW
NRC = N // RH
EBATCH = 16384
NBATCH = E // EBATCH
NG = EBATCH // 16


def _make_adj():
    mesh = plsc.VectorSubcoreMesh(core_axis_name="c", subcore_axis_name="s",
                                  num_cores=NC, num_subcores=NS)

    @functools.partial(
        pl.kernel,
        out_type=jax.ShapeDtypeStruct((N, N), jnp.float32),
        mesh=mesh,
        scratch_types=[
            pltpu.VMEM((EBATCH,), jnp.int32),
            pltpu.VMEM((EBATCH,), jnp.int32),
            pltpu.VMEM((RH, CW), jnp.float32),
        ],
        compiler_params=pltpu.CompilerParams(needs_layout_passes=False),
    )
    def adj(ei, zeros, out, src_v, dst_v, acc_v):
        cid = lax.axis_index("c")
        sid = lax.axis_index("s")
        wid = cid * NS + sid
        ones16 = jnp.ones((16,), jnp.float32)
        for p in range(2):
            combo = wid + 32 * p
            clo = pl.multiple_of((combo % NCC) * CW, CW)
            rlo = pl.multiple_of((combo // NCC) * RH, RH)
            pltpu.sync_copy(zeros, acc_v)
            for b in range(NBATCH):
                base = pl.multiple_of(b * EBATCH, EBATCH)
                pltpu.sync_copy(ei.at[0, pl.ds(base, EBATCH)], src_v)
                pltpu.sync_copy(ei.at[1, pl.ds(base, EBATCH)], dst_v)

                @plsc.parallel_loop(0, NG, unroll=8)
                def _(g):
                    s16 = src_v[pl.ds(g * 16, 16)]
                    d16 = dst_v[pl.ds(g * 16, 16)]
                    sc = s16 - clo
                    dr = d16 - rlo
                    m = ((sc >= 0) & (sc < CW)) & ((dr >= 0) & (dr < RH))
                    plsc.addupdate_scatter(acc_v, [dr, sc], ones16, mask=m)
            pltpu.sync_copy(acc_v, out.at[pl.ds(rlo, RH), pl.ds(clo, CW)])

    return adj


_adj_cache = {}


def _adj():
    if "adj" not in _adj_cache:
        _adj_cache["adj"] = _make_adj()
    return _adj_cache["adj"]


def _relu(v):
    return jnp.maximum(v, 0.0)


def _dot(a, b):
    return jnp.dot(a, b, preferred_element_type=jnp.float32)


_VMEM_PARAMS = pltpu.CompilerParams(vmem_limit_bytes=100 * 1024 * 1024)


def _tp_body(adj, abf_o, cnt_o):
    abf_o[:] = adj[:].astype(jnp.bfloat16)
    cnt_o[:] = jnp.sum(adj[:], axis=1, keepdims=True)


def _t1_body(abf, cnt, x, wl, wr, b, g1w, u1_o, dinv_o):
    s = _dot(abf[:], x[:].astype(jnp.bfloat16))
    agg = s / jnp.maximum(cnt[:], 1.0)
    h1 = _relu(_dot(agg, wl[:]) + _dot(x[:], wr[:]) + b[:])
    dinv = lax.rsqrt(cnt[:] + 1.0)
    dinv_o[:] = dinv
    u1_o[:] = dinv * _dot(h1, g1w[:])


def _t2_body(abf, u1, dinv, g1b, g2w, u2_o):
    s = _dot(abf[:], u1[:].astype(jnp.bfloat16))
    h2 = _relu(dinv[:] * (s + u1[:]) + g1b[:])
    u2_o[:] = dinv[:] * _dot(h2, g2w[:])


CB = 1024


def _t3_body(abf, u2, dinv, g2b, f1w, f1b, f2w, f2b, ow, ob, out_o, h5):
    @pl.when(pl.program_id(0) == 0)
    def _():
        s = _dot(abf[:], u2[:].astype(jnp.bfloat16))
        h3 = _relu(dinv[:] * (s + u2[:]) + g2b[:])
        h4 = _relu(_dot(h3, f1w[:]) + f1b[:])
        h5[:] = _relu(_dot(h4, f2w[:]) + f2b[:])

    out_o[:] = _dot(h5[:], ow[:]) + ob[:]


def _full2(i):
    del i
    return 0, 0


def _tp_call(adj):
    return pl.pallas_call(
        _tp_body,
        out_shape=(jax.ShapeDtypeStruct((N, N), jnp.bfloat16),
                   jax.ShapeDtypeStruct((N, 1), jnp.float32)),
        compiler_params=_VMEM_PARAMS,
    )(adj)


def _t1_call(abf, cnt, x, wl, wr, b, g1w):
    return pl.pallas_call(
        _t1_body,
        out_shape=(jax.ShapeDtypeStruct((N, 2 * H), jnp.float32),
                   jax.ShapeDtypeStruct((N, 1), jnp.float32)),
        compiler_params=_VMEM_PARAMS,
    )(abf, cnt, x, wl, wr, b, g1w)


def _t2_call(abf, u1, dinv, g1b, g2w):
    return pl.pallas_call(
        _t2_body,
        out_shape=jax.ShapeDtypeStruct((N, H), jnp.float32),
        compiler_params=_VMEM_PARAMS,
    )(abf, u1, dinv, g1b, g2w)


def _t3_call(abf, u2, dinv, g2b, f1w, f1b, f2w, f2b, ow, ob):
    nblk = 4 * N // CB
    return pl.pallas_call(
        _t3_body,
        grid=(nblk,),
        in_specs=[
            pl.BlockSpec((N, N), _full2),
            pl.BlockSpec((N, H), _full2),
            pl.BlockSpec((N, 1), _full2),
            pl.BlockSpec((1, H), _full2),
            pl.BlockSpec((H, H), _full2),
            pl.BlockSpec((1, H), _full2),
            pl.BlockSpec((H, H), _full2),
            pl.BlockSpec((1, H), _full2),
            pl.BlockSpec((H, CB), lambda i: (0, i)),
            pl.BlockSpec((1, CB), lambda i: (0, i)),
        ],
        out_specs=pl.BlockSpec((N, CB), lambda i: (0, i)),
        out_shape=jax.ShapeDtypeStruct((N, 4 * N), jnp.float32),
        scratch_shapes=[pltpu.VMEM((N, H), jnp.float32)],
        compiler_params=_VMEM_PARAMS,
    )(abf, u2, dinv, g2b, f1w, f1b, f2w, f2b, ow, ob)


def kernel(x, edge_index, sage_wl, sage_wr, sage_b, gcn1_w, gcn1_b,
           gcn2_w, gcn2_b, fc1_w, fc1_b, fc2_w, fc2_b, out_w, out_b):
    adj = _adj()(edge_index, jnp.zeros((RH, CW), jnp.float32))
    abf, cnt = _tp_call(adj)
    u1, dinv = _t1_call(abf, cnt, x, sage_wl, sage_wr,
                        sage_b.reshape(1, 2 * H), gcn1_w)
    u2 = _t2_call(abf, u1, dinv, gcn1_b.reshape(1, 2 * H), gcn2_w)
    out = _t3_call(abf, u2, dinv, gcn2_b.reshape(1, H),
                   fc1_w, fc1_b.reshape(1, H), fc2_w, fc2_b.reshape(1, H),
                   out_w, out_b.reshape(1, 4 * N))
    return out.reshape(N, 4, N)


# speedup vs baseline: 1.0189x; 1.0189x over previous
import functools

import jax
import jax.numpy as jnp
from jax import lax
from jax.experimental import pallas as pl
from jax.experimental.pallas import tpu as pltpu
from jax.experimental.pallas import tpu_sc as plsc

N = 2048
E = 131072
D = 128
H = 128

NC = 2
NS = 16
CW = 128
RH = 512
NCC = N // CW
NRC = N // RH
EBATCH = 16384
NBATCH = E // EBATCH
NG = EBATCH // 16


def _make_adj():
    mesh = plsc.VectorSubcoreMesh(core_axis_name="c", subcore_axis_name="s",
                                  num_cores=NC, num_subcores=NS)

    @functools.partial(
        pl.kernel,
        out_type=jax.ShapeDtypeStruct((N, N), jnp.float32),
        mesh=mesh,
        scratch_types=[
            pltpu.VMEM((EBATCH,), jnp.int32),
            pltpu.VMEM((EBATCH,), jnp.int32),
            pltpu.VMEM((RH, CW), jnp.float32),
        ],
        compiler_params=pltpu.CompilerParams(needs_layout_passes=False),
    )
    def adj(ei, zeros, out, src_v, dst_v, acc_v):
        cid = lax.axis_index("c")
        sid = lax.axis_index("s")
        wid = cid * NS + sid
        ones16 = jnp.ones((16,), jnp.float32)
        for p in range(2):
            combo = wid + 32 * p
            clo = pl.multiple_of((combo % NCC) * CW, CW)
            rlo = pl.multiple_of((combo // NCC) * RH, RH)
            pltpu.sync_copy(zeros, acc_v)
            for b in range(NBATCH):
                base = pl.multiple_of(b * EBATCH, EBATCH)
                pltpu.sync_copy(ei.at[0, pl.ds(base, EBATCH)], src_v)
                pltpu.sync_copy(ei.at[1, pl.ds(base, EBATCH)], dst_v)

                @plsc.parallel_loop(0, NG, unroll=8)
                def _(g):
                    s16 = src_v[pl.ds(g * 16, 16)]
                    d16 = dst_v[pl.ds(g * 16, 16)]
                    sc = s16 - clo
                    dr = d16 - rlo
                    m = ((sc >= 0) & (sc < CW)) & ((dr >= 0) & (dr < RH))
                    plsc.addupdate_scatter(acc_v, [dr, sc], ones16, mask=m)
            pltpu.sync_copy(acc_v, out.at[pl.ds(rlo, RH), pl.ds(clo, CW)])

    return adj


_adj_cache = {}


def _adj():
    if "adj" not in _adj_cache:
        _adj_cache["adj"] = _make_adj()
    return _adj_cache["adj"]


def _relu(v):
    return jnp.maximum(v, 0.0)


def _dot(a, b):
    return jnp.dot(a, b, preferred_element_type=jnp.float32)


_VMEM_PARAMS = pltpu.CompilerParams(vmem_limit_bytes=100 * 1024 * 1024)


def _tp_body(adj, ow, abf_o, cnt_o, owbf_o):
    abf_o[:] = adj[:].astype(jnp.bfloat16)
    cnt_o[:] = jnp.sum(adj[:], axis=1, keepdims=True)
    owbf_o[:] = ow[:].astype(jnp.bfloat16)


CB = 1024


def _tf_body(abf, cnt, x, wl, wr, b, g1w, g1b, g2w, g2b,
             f1w, f1b, f2w, f2b, owbf, ob, out_o, h5bf):
    @pl.when(pl.program_id(0) == 0)
    def _():
        s = _dot(abf[:], x[:].astype(jnp.bfloat16))
        agg = s / jnp.maximum(cnt[:], 1.0)
        h1 = _relu(_dot(agg, wl[:]) + _dot(x[:], wr[:]) + b[:])
        dinv = lax.rsqrt(cnt[:] + 1.0)
        u1 = dinv * _dot(h1, g1w[:])
        s1 = _dot(abf[:], u1.astype(jnp.bfloat16))
        h2 = _relu(dinv * (s1 + u1) + g1b[:])
        u2 = dinv * _dot(h2, g2w[:])
        s2 = _dot(abf[:], u2.astype(jnp.bfloat16))
        h3 = _relu(dinv * (s2 + u2) + g2b[:])
        h4 = _relu(_dot(h3, f1w[:]) + f1b[:])
        h5bf[:] = _relu(_dot(h4, f2w[:]) + f2b[:]).astype(jnp.bfloat16)

    out_o[:] = _dot(h5bf[:], owbf[:]) + ob[:]


def _full2(i):
    del i
    return 0, 0


def _tp_call(adj, ow):
    return pl.pallas_call(
        _tp_body,
        out_shape=(jax.ShapeDtypeStruct((N, N), jnp.bfloat16),
                   jax.ShapeDtypeStruct((N, 1), jnp.float32),
                   jax.ShapeDtypeStruct((H, 4 * N), jnp.bfloat16)),
        compiler_params=_VMEM_PARAMS,
    )(adj, ow)


def _tf_call(abf, cnt, x, wl, wr, b, g1w, g1b, g2w, g2b,
             f1w, f1b, f2w, f2b, owbf, ob):
    nblk = 4 * N // CB
    small = [
        pl.BlockSpec((N, 1), _full2),
        pl.BlockSpec((N, H), _full2),
        pl.BlockSpec((H, 2 * H), _full2),
        pl.BlockSpec((H, 2 * H), _full2),
        pl.BlockSpec((1, 2 * H), _full2),
        pl.BlockSpec((2 * H, 2 * H), _full2),
        pl.BlockSpec((1, 2 * H), _full2),
        pl.BlockSpec((2 * H, H), _full2),
        pl.BlockSpec((1, H), _full2),
        pl.BlockSpec((H, H), _full2),
        pl.BlockSpec((1, H), _full2),
        pl.BlockSpec((H, H), _full2),
        pl.BlockSpec((1, H), _full2),
    ]
    return pl.pallas_call(
        _tf_body,
        grid=(nblk,),
        in_specs=[pl.BlockSpec((N, N), _full2)] + small + [
            pl.BlockSpec((H, CB), lambda i: (0, i)),
            pl.BlockSpec((1, CB), lambda i: (0, i)),
        ],
        out_specs=pl.BlockSpec((N, CB), lambda i: (0, i)),
        out_shape=jax.ShapeDtypeStruct((N, 4 * N), jnp.float32),
        scratch_shapes=[pltpu.VMEM((N, H), jnp.bfloat16)],
        compiler_params=_VMEM_PARAMS,
    )(abf, cnt, x, wl, wr, b, g1w, g1b, g2w, g2b,
      f1w, f1b, f2w, f2b, owbf, ob)


def kernel(x, edge_index, sage_wl, sage_wr, sage_b, gcn1_w, gcn1_b,
           gcn2_w, gcn2_b, fc1_w, fc1_b, fc2_w, fc2_b, out_w, out_b):
    adj = _adj()(edge_index, jnp.zeros((RH, CW), jnp.float32))
    abf, cnt, owbf = _tp_call(adj, out_w)
    out = _tf_call(abf, cnt, x, sage_wl, sage_wr, sage_b.reshape(1, 2 * H),
                   gcn1_w, gcn1_b.reshape(1, 2 * H),
                   gcn2_w, gcn2_b.reshape(1, H),
                   fc1_w, fc1_b.reshape(1, H), fc2_w, fc2_b.reshape(1, H),
                   owbf, out_b.reshape(1, 4 * N))
    return out.reshape(N, 4, N)


# speedup vs baseline: 1.1009x; 1.0805x over previous
import functools

import jax
import jax.numpy as jnp
from jax import lax
from jax.experimental import pallas as pl
from jax.experimental.pallas import tpu as pltpu
from jax.experimental.pallas import tpu_sc as plsc

N = 2048
E = 131072
D = 128
H = 128

NC = 2
NS = 16
CW = 128
RH = 512
NCC = N // CW
NRC = N // RH
EBATCH = 8192
NBATCH = E // EBATCH
NG = EBATCH // 16


def _make_adj():
    mesh = plsc.VectorSubcoreMesh(core_axis_name="c", subcore_axis_name="s",
                                  num_cores=NC, num_subcores=NS)

    @functools.partial(
        pl.kernel,
        out_type=jax.ShapeDtypeStruct((N, N), jnp.float32),
        mesh=mesh,
        scratch_types=[
            pltpu.VMEM((2, EBATCH), jnp.int32),
            pltpu.VMEM((2, EBATCH), jnp.int32),
            pltpu.VMEM((RH, CW), jnp.float32),
            pltpu.SemaphoreType.DMA,
            pltpu.SemaphoreType.DMA,
        ],
        compiler_params=pltpu.CompilerParams(needs_layout_passes=False),
    )
    def adj(ei, zeros, out, src_v, dst_v, acc_v, sem_s, sem_d):
        cid = lax.axis_index("c")
        sid = lax.axis_index("s")
        wid = cid * NS + sid
        ones16 = jnp.ones((16,), jnp.float32)

        def start(b):
            base = pl.multiple_of(b * EBATCH, EBATCH)
            ds = pltpu.async_copy(ei.at[0, pl.ds(base, EBATCH)],
                                  src_v.at[b % 2], sem_s)
            dd = pltpu.async_copy(ei.at[1, pl.ds(base, EBATCH)],
                                  dst_v.at[b % 2], sem_d)
            return ds, dd

        for p in range(2):
            combo = wid + 32 * p
            clo = pl.multiple_of((combo % NCC) * CW, CW)
            rlo = pl.multiple_of((combo // NCC) * RH, RH)
            pend = start(0)
            pltpu.sync_copy(zeros, acc_v)
            for b in range(NBATCH):
                cur = b % 2
                pend[0].wait()
                pend[1].wait()
                if b + 1 < NBATCH:
                    pend = start(b + 1)

                @plsc.parallel_loop(0, NG, unroll=8)
                def _(g):
                    s16 = src_v[cur, pl.ds(g * 16, 16)]
                    d16 = dst_v[cur, pl.ds(g * 16, 16)]
                    sc = s16 - clo
                    dr = d16 - rlo
                    m = ((sc >= 0) & (sc < CW)) & ((dr >= 0) & (dr < RH))
                    plsc.addupdate_scatter(acc_v, [dr, sc], ones16, mask=m)
            pltpu.sync_copy(acc_v, out.at[pl.ds(rlo, RH), pl.ds(clo, CW)])

    return adj


_adj_cache = {}


def _adj():
    if "adj" not in _adj_cache:
        _adj_cache["adj"] = _make_adj()
    return _adj_cache["adj"]


def _relu(v):
    return jnp.maximum(v, 0.0)


def _dot(a, b):
    return jnp.dot(a, b, preferred_element_type=jnp.float32)


_VMEM_PARAMS = pltpu.CompilerParams(vmem_limit_bytes=100 * 1024 * 1024)


def _tp_body(adj, ow, abf_o, cnt_o, owbf_o):
    abf_o[:] = adj[:].astype(jnp.bfloat16)
    cnt_o[:] = jnp.sum(adj[:], axis=1, keepdims=True)
    owbf_o[:] = ow[:].astype(jnp.bfloat16)


CB = 1024


def _tf_body(abf, cnt, x, wl, wr, b, g1w, g1b, g2w, g2b,
             f1w, f1b, f2w, f2b, owbf, ob, out_o, h5bf):
    @pl.when(pl.program_id(0) == 0)
    def _():
        s = _dot(abf[:], x[:].astype(jnp.bfloat16))
        agg = s / jnp.maximum(cnt[:], 1.0)
        h1 = _relu(_dot(agg, wl[:]) + _dot(x[:], wr[:]) + b[:])
        dinv = lax.rsqrt(cnt[:] + 1.0)
        u1 = dinv * _dot(h1, g1w[:])
        s1 = _dot(abf[:], u1.astype(jnp.bfloat16))
        h2 = _relu(dinv * (s1 + u1) + g1b[:])
        u2 = dinv * _dot(h2, g2w[:])
        s2 = _dot(abf[:], u2.astype(jnp.bfloat16))
        h3 = _relu(dinv * (s2 + u2) + g2b[:])
        h4 = _relu(_dot(h3, f1w[:]) + f1b[:])
        h5bf[:] = _relu(_dot(h4, f2w[:]) + f2b[:]).astype(jnp.bfloat16)

    out_o[:] = _dot(h5bf[:], owbf[:]) + ob[:]


def _full2(i):
    del i
    return 0, 0


def _tp_call(adj, ow):
    return pl.pallas_call(
        _tp_body,
        out_shape=(jax.ShapeDtypeStruct((N, N), jnp.bfloat16),
                   jax.ShapeDtypeStruct((N, 1), jnp.float32),
                   jax.ShapeDtypeStruct((H, 4 * N), jnp.bfloat16)),
        compiler_params=_VMEM_PARAMS,
    )(adj, ow)


def _tf_call(abf, cnt, x, wl, wr, b, g1w, g1b, g2w, g2b,
             f1w, f1b, f2w, f2b, owbf, ob):
    nblk = 4 * N // CB
    small = [
        pl.BlockSpec((N, 1), _full2),
        pl.BlockSpec((N, H), _full2),
        pl.BlockSpec((H, 2 * H), _full2),
        pl.BlockSpec((H, 2 * H), _full2),
        pl.BlockSpec((1, 2 * H), _full2),
        pl.BlockSpec((2 * H, 2 * H), _full2),
        pl.BlockSpec((1, 2 * H), _full2),
        pl.BlockSpec((2 * H, H), _full2),
        pl.BlockSpec((1, H), _full2),
        pl.BlockSpec((H, H), _full2),
        pl.BlockSpec((1, H), _full2),
        pl.BlockSpec((H, H), _full2),
        pl.BlockSpec((1, H), _full2),
    ]
    return pl.pallas_call(
        _tf_body,
        grid=(nblk,),
        in_specs=[pl.BlockSpec((N, N), _full2)] + small + [
            pl.BlockSpec((H, CB), lambda i: (0, i)),
            pl.BlockSpec((1, CB), lambda i: (0, i)),
        ],
        out_specs=pl.BlockSpec((N, CB), lambda i: (0, i)),
        out_shape=jax.ShapeDtypeStruct((N, 4 * N), jnp.float32),
        scratch_shapes=[pltpu.VMEM((N, H), jnp.bfloat16)],
        compiler_params=_VMEM_PARAMS,
    )(abf, cnt, x, wl, wr, b, g1w, g1b, g2w, g2b,
      f1w, f1b, f2w, f2b, owbf, ob)


def kernel(x, edge_index, sage_wl, sage_wr, sage_b, gcn1_w, gcn1_b,
           gcn2_w, gcn2_b, fc1_w, fc1_b, fc2_w, fc2_b, out_w, out_b):
    adj = _adj()(edge_index, jnp.zeros((RH, CW), jnp.float32))
    abf, cnt, owbf = _tp_call(adj, out_w)
    out = _tf_call(abf, cnt, x, sage_wl, sage_wr, sage_b.reshape(1, 2 * H),
                   gcn1_w, gcn1_b.reshape(1, 2 * H),
                   gcn2_w, gcn2_b.reshape(1, H),
                   fc1_w, fc1_b.reshape(1, H), fc2_w, fc2_b.reshape(1, H),
                   owbf, out_b.reshape(1, 4 * N))
    return out.reshape(N, 4, N)
